# Initial kernel scaffold; baseline (speedup 1.0000x reference)
import jax
import jax.numpy as jnp
from jax.experimental import pallas as pl


def kernel(img_embs, cap_embs, cap_lens, ln_g, ln_b, w1, b1, w2, b2, scale):
    raise NotImplementedError("write your pallas kernel here")


# speedup vs baseline: 4.2158x; 4.2158x over previous
import math

import jax
import jax.numpy as jnp
from jax.experimental import pallas as pl
from jax.experimental.pallas import tpu as pltpu

_EPS = 1e-12
_NEG = -1e30


def _erf(x):
    return jax.lax.erf(x)


def _body(nkeep, cls_ref, sp_ref, cap_ref, lens_ref, lng_ref, lnb_ref,
          w1_ref, b1_ref, w2_ref, b2_ref, scale_ref, out_ref,
          sc_ref, keep_ref):
    f32 = jnp.float32
    cls = cls_ref[0]
    sp = sp_ref[0]
    caps = cap_ref[...]
    lens = lens_ref[...]
    g = lng_ref[...]
    bta = lnb_ref[...]
    w1 = w1_ref[...]
    b1 = b1_ref[...]
    w2 = w2_ref[...]
    b2 = b2_ref[...]
    scale = scale_ref[0, 0]

    L = sp.shape[0]
    T, LW, C = caps.shape
    K = w2.shape[1]

    cls_n = cls / jnp.maximum(
        jnp.sqrt(jnp.sum(cls * cls, axis=1, keepdims=True)), _EPS)
    sp_n = sp / jnp.maximum(
        jnp.sqrt(jnp.sum(sp * sp, axis=1, keepdims=True)), _EPS)
    self_attn = jnp.sum(sp_n * cls_n, axis=1, keepdims=True)

    m = jnp.mean(sp, axis=1, keepdims=True)
    xc = sp - m
    v = jnp.mean(xc * xc, axis=1, keepdims=True)
    ln = xc / jnp.sqrt(v + 1e-5) * g + bta
    h = jnp.dot(ln, w1, preferred_element_type=f32) + b1
    h = 0.5 * h * (1.0 + _erf(h / jnp.sqrt(jnp.float32(2.0))))
    logits = jnp.dot(h, w2, preferred_element_type=f32) + b2
    lgT = (logits * scale).T

    cap_ss = jnp.sum(caps * caps, axis=2, keepdims=True)
    cn = caps / jnp.maximum(jnp.sqrt(cap_ss), _EPS)
    cap_glo = cn[:, 0, :]
    cap_attn = jnp.dot(cap_glo, sp_n.T, preferred_element_type=f32)
    scores = cap_attn + self_attn.T

    sc_ref[...] = scores
    ii = jax.lax.broadcasted_iota(jnp.int32, (L, L), 0)
    jj = jax.lax.broadcasted_iota(jnp.int32, (L, L), 1)
    tie_low = jj < ii

    def _rank_step(t, _):
        row = sc_ref[pl.ds(t, 1), :]
        col = jnp.transpose(row)
        rj = jnp.broadcast_to(row, (L, L))
        beats = jnp.logical_or(rj > col,
                               jnp.logical_and(rj == col, tie_low))
        rank = jnp.sum(beats.astype(f32), axis=1, keepdims=True)
        keep_ref[pl.ds(t, 1), :] = jnp.transpose(
            (rank < jnp.float32(nkeep)).astype(f32))
        return 0

    jax.lax.fori_loop(0, T, _rank_step, 0)
    keep = keep_ref[...] > 0.5

    sc_non = jnp.where(keep, _NEG, scores)
    mn = jnp.max(sc_non, axis=1, keepdims=True)
    pn = jnp.exp(sc_non - mn)
    pn = pn / jnp.sum(pn, axis=1, keepdims=True)

    ml = jnp.where(keep[:, None, :], lgT[None], _NEG)
    mm = jnp.max(ml, axis=2, keepdims=True)
    wt = jnp.exp(ml - mm)
    wt = wt / jnp.sum(wt, axis=2, keepdims=True)

    wfull = jnp.concatenate([wt, pn[:, None, :]], axis=1)
    rows = jnp.dot(wfull.reshape(T * (K + 1), L), sp,
                   preferred_element_type=f32)
    rn = rows / jnp.maximum(
        jnp.sqrt(jnp.sum(rows * rows, axis=1, keepdims=True)), _EPS)
    rn3 = rn.reshape(T, K + 1, C)

    sim_cls = jnp.dot(cn.reshape(T * LW, C), cls_n.T,
                      preferred_element_type=f32).reshape(T, LW)
    sim_rows = jax.lax.dot_general(
        cn, rn3, (((2,), (2,)), ((0,), (0,))),
        preferred_element_type=f32)
    simmax = jnp.maximum(jnp.max(sim_rows, axis=2), sim_cls)

    widx = jax.lax.broadcasted_iota(jnp.int32, (T, LW), 1).astype(f32)
    ssum = jnp.sum(jnp.where(widx < lens, simmax, 0.0), axis=1)
    out_ref[0, 0, :] = ssum / lens[:, 0]


def kernel(img_embs, cap_embs, cap_lens, ln_g, ln_b, w1, b1, w2, b2, scale):
    B_v, L_v, C = img_embs.shape
    T, LW, _ = cap_embs.shape
    H = w1.shape[1]
    K = w2.shape[1]
    L = L_v - 1
    nkeep = math.ceil(L * 0.6)
    f32 = jnp.float32

    cls_all = img_embs[:, 0:1, :]
    sp_all = img_embs[:, 1:, :]
    lens_f = cap_lens.astype(f32).reshape(T, 1)
    g2 = ln_g.reshape(1, C)
    b2d = ln_b.reshape(1, C)
    b1_2 = b1.reshape(1, H)
    b2_2 = b2.reshape(1, K)
    sc2 = scale.reshape(1, 1)

    import functools
    body = functools.partial(_body, nkeep)

    out3 = pl.pallas_call(
        body,
        grid=(B_v,),
        in_specs=[
            pl.BlockSpec((1, 1, C), lambda b: (b, 0, 0)),
            pl.BlockSpec((1, L, C), lambda b: (b, 0, 0)),
            pl.BlockSpec((T, LW, C), lambda b: (0, 0, 0)),
            pl.BlockSpec((T, 1), lambda b: (0, 0)),
            pl.BlockSpec((1, C), lambda b: (0, 0)),
            pl.BlockSpec((1, C), lambda b: (0, 0)),
            pl.BlockSpec((C, H), lambda b: (0, 0)),
            pl.BlockSpec((1, H), lambda b: (0, 0)),
            pl.BlockSpec((H, K), lambda b: (0, 0)),
            pl.BlockSpec((1, K), lambda b: (0, 0)),
            pl.BlockSpec((1, 1), lambda b: (0, 0)),
        ],
        out_specs=pl.BlockSpec((1, 1, T), lambda b: (b, 0, 0)),
        out_shape=jax.ShapeDtypeStruct((B_v, 1, T), f32),
        scratch_shapes=[
            pltpu.VMEM((T, L), f32),
            pltpu.VMEM((T, L), f32),
        ],
    )(cls_all, sp_all, cap_embs, lens_f, g2, b2d, w1, b1_2, w2, b2_2, sc2)
    return out3.reshape(B_v, T)
